# baseline (speedup 1.0000x reference)
import jax
import jax.numpy as jnp
from jax import lax
from jax.experimental import pallas as pl
from jax.experimental.pallas import tpu as pltpu
from jax.experimental.pallas import tpu_sc as plsc

N_NODES = 10000
N_EDGES = 160000
D_IN = 256
D_OUT = 256
DH = 128
NC = 2
NS = 16
LANES = 16
EPT = N_EDGES // NS
K = 80
NB = EPT // K
SB = 25
NSB = NB // SB
SBE = SB * K
NSLOT = 4
T = SB // NSLOT
ROWS_PER_TILE = N_NODES // NS
ZROWS = 125


def _sc_agg_body(x2_hbm, src_hbm, dst4_hbm, attr_hbm, out_hbm,
                 gidx, dstb, attrb, rows, agg_sh,
                 gsem0, gsem1, gsem2, gsem3):
    c = lax.axis_index("c")
    sid = lax.axis_index("s")
    ebase = sid * EPT

    @pl.loop(0, NSLOT * K)
    def _zero(r):
        for j in range(DH // LANES):
            rows[r, pl.ds(j * LANES, LANES)] = jnp.zeros((LANES,), jnp.float32)

    zb = sid * ROWS_PER_TILE
    pltpu.sync_copy(rows, agg_sh.at[pl.ds(zb, NSLOT * K)])
    pltpu.sync_copy(rows.at[pl.ds(0, ROWS_PER_TILE - NSLOT * K)],
                    agg_sh.at[pl.ds(zb + NSLOT * K, ROWS_PER_TILE - NSLOT * K)])

    plsc.subcore_barrier()

    def gather_start(b, slot, sem):
        pltpu.async_copy(x2_hbm.at[gidx.at[pl.ds(b * K, K)]],
                         rows.at[pl.ds(slot * K, K)], sem)

    def gather_wait(b, slot, sem):
        pltpu.make_async_copy(x2_hbm.at[gidx.at[pl.ds(b * K, K)]],
                              rows.at[pl.ds(slot * K, K)], sem).wait()

    def scatter_sync(b, slot):
        pltpu.sync_copy(rows.at[pl.ds(slot * K, K)],
                        agg_sh.at[dstb.at[b, 0]], add=True)

    def scale(b, slot):
        @pl.loop(0, K // LANES)
        def _scale(q):
            a16 = attrb[pl.ds(b * K + q * LANES, LANES)]
            for e in range(LANES):
                av = jnp.full((LANES,), a16[e], jnp.float32)
                r = slot * K + q * LANES + e
                for j in range(DH // LANES):
                    sl = pl.ds(j * LANES, LANES)
                    rows[r, sl] = rows[r, sl] * av


    @pl.loop(0, NSB)
    def _super(s):
        pltpu.sync_copy(src_hbm.at[pl.ds(ebase + s * SBE, SBE)], gidx)
        pltpu.sync_copy(attr_hbm.at[pl.ds(ebase + s * SBE, SBE)], attrb)
        pltpu.sync_copy(dst4_hbm.at[sid, s], dstb)

        @pl.loop(0, SBE // LANES)
        def _mkidx(j):
            sl = pl.ds(j * LANES, LANES)
            gidx[sl] = gidx[sl] * 2 + c

        sems = (gsem0, gsem1, gsem2, gsem3)
        for l in range(NSLOT):
            gather_start(l, l, sems[l])

        @pl.loop(0, T)
        def _quad(t):
            u = NSLOT * t
            for l in range(NSLOT):
                gather_wait(u + l, l, sems[l])
                scale(u + l, l)
                scatter_sync(u + l, l)

                @pl.when(u + l + NSLOT < SB)
                def _pref():
                    gather_start(u + l + NSLOT, l, sems[l])

        gather_wait(SB - 1, 0, gsem0)
        scale(SB - 1, 0)
        scatter_sync(SB - 1, 0)

    plsc.subcore_barrier()

    w = c * NS + sid
    pltpu.sync_copy(agg_sh.at[pl.ds(sid * ROWS_PER_TILE, ROWS_PER_TILE)],
                    out_hbm.at[w])


def _sc_agg(x2, src, dst4, attr):
    mesh = plsc.VectorSubcoreMesh(core_axis_name="c", subcore_axis_name="s")
    kern = pl.kernel(
        _sc_agg_body,
        out_type=jax.ShapeDtypeStruct((NC * NS, ROWS_PER_TILE, DH),
                                      jnp.float32),
        mesh=mesh,
        scratch_types=[
            pltpu.VMEM((SBE,), jnp.int32),
            pltpu.VMEM((SB, 1, K), jnp.int32),
            pltpu.VMEM((SBE,), jnp.float32),
            pltpu.VMEM((NSLOT * K, DH), jnp.float32),
            pltpu.VMEM_SHARED((N_NODES, DH), jnp.float32),
            pltpu.SemaphoreType.DMA,
            pltpu.SemaphoreType.DMA,
            pltpu.SemaphoreType.DMA,
            pltpu.SemaphoreType.DMA,
        ],
    )
    return kern(x2, src, dst4, attr)


def _tc_body(x_ref, a0_ref, a1_ref, wroot_ref, wr0_ref, wr1_ref, b_ref,
             o_ref):
    acc = jnp.dot(x_ref[0], wroot_ref[...],
                  preferred_element_type=jnp.float32)
    acc += jnp.dot(a0_ref[0], wr0_ref[...],
                   preferred_element_type=jnp.float32)
    acc += jnp.dot(a1_ref[0], wr1_ref[...],
                   preferred_element_type=jnp.float32)
    o_ref[0] = jnp.maximum(acc + b_ref[...], 0.0)


def _tc_linear(x3, agg, wroot_t, wr0, wr1, b2):
    grid = (NS,)
    rpt = ROWS_PER_TILE
    return pl.pallas_call(
        _tc_body,
        grid=grid,
        in_specs=[
            pl.BlockSpec((1, rpt, D_IN), lambda i: (i, 0, 0)),
            pl.BlockSpec((1, rpt, DH), lambda i: (i, 0, 0)),
            pl.BlockSpec((1, rpt, DH), lambda i: (i + NS, 0, 0)),
            pl.BlockSpec((D_IN, D_OUT), lambda i: (0, 0)),
            pl.BlockSpec((DH, D_OUT), lambda i: (0, 0)),
            pl.BlockSpec((DH, D_OUT), lambda i: (0, 0)),
            pl.BlockSpec((1, D_OUT), lambda i: (0, 0)),
        ],
        out_specs=pl.BlockSpec((1, rpt, D_OUT), lambda i: (i, 0, 0)),
        out_shape=jax.ShapeDtypeStruct((NS, rpt, D_OUT), jnp.float32),
    )(x3, agg, agg, wroot_t, wr0, wr1, b2)


@jax.jit
def kernel(x, edge_index, edge_attr, W_rel, b_rel, W_root):
    src = edge_index[0].astype(jnp.int32)
    dst = edge_index[1].astype(jnp.int32)
    x2 = x.reshape(2 * N_NODES, DH)
    dst4 = dst.reshape(NS, NSB, SB, 1, K)

    agg = _sc_agg(x2, src, dst4, edge_attr)
    x3 = x.reshape(NS, ROWS_PER_TILE, D_IN)

    wroot_t = W_root.T
    wr0 = W_rel[:, :DH].T
    wr1 = W_rel[:, DH:].T
    b2 = b_rel[None, :]
    out3 = _tc_linear(x3, agg, wroot_t, wr0, wr1, b2)
    return out3.reshape(N_NODES, D_OUT)


# speedup vs baseline: 1.0121x; 1.0121x over previous
import jax
import jax.numpy as jnp
from jax import lax
from jax.experimental import pallas as pl
from jax.experimental.pallas import tpu as pltpu
from jax.experimental.pallas import tpu_sc as plsc

N_NODES = 10000
N_EDGES = 160000
D_IN = 256
D_OUT = 256
DH = 128
NC = 2
NS = 16
LANES = 16
EPT = N_EDGES // NS
K = 80
NB = EPT // K
SB = 25
NSB = NB // SB
SBE = SB * K
NSLOT = 4
T = SB // NSLOT
ROWS_PER_TILE = N_NODES // NS
ZROWS = 125


def _sc_agg_body(x2_hbm, src_hbm, dst4_hbm, attr_hbm, out_hbm,
                 gidx, dstb, attrb, rows, agg_sh,
                 gsem0, gsem1, gsem2, gsem3):
    c = lax.axis_index("c")
    sid = lax.axis_index("s")
    ebase = sid * EPT

    @pl.loop(0, NSLOT * K)
    def _zero(r):
        for j in range(DH // LANES):
            rows[r, pl.ds(j * LANES, LANES)] = jnp.zeros((LANES,), jnp.float32)

    zb = sid * ROWS_PER_TILE
    pltpu.sync_copy(rows, agg_sh.at[pl.ds(zb, NSLOT * K)])
    pltpu.sync_copy(rows.at[pl.ds(0, ROWS_PER_TILE - NSLOT * K)],
                    agg_sh.at[pl.ds(zb + NSLOT * K, ROWS_PER_TILE - NSLOT * K)])

    plsc.subcore_barrier()

    def gather_start(b, slot, sem):
        pltpu.async_copy(x2_hbm.at[gidx.at[pl.ds(b * K, K)]],
                         rows.at[pl.ds(slot * K, K)], sem)

    def gather_wait(b, slot, sem):
        pltpu.make_async_copy(x2_hbm.at[gidx.at[pl.ds(b * K, K)]],
                              rows.at[pl.ds(slot * K, K)], sem).wait()

    def scatter_sync(b, slot):
        pltpu.sync_copy(rows.at[pl.ds(slot * K, K)],
                        agg_sh.at[dstb.at[b, 0]], add=True)

    def scale(b, slot):
        @pl.loop(0, K // LANES)
        def _scale(q):
            a16 = attrb[pl.ds(b * K + q * LANES, LANES)]
            for e in range(LANES):
                av = jnp.full((LANES,), a16[e], jnp.float32)
                r = slot * K + q * LANES + e
                for j in range(DH // LANES):
                    sl = pl.ds(j * LANES, LANES)
                    rows[r, sl] = rows[r, sl] * av


    @pl.loop(0, NSB)
    def _super(s):
        pltpu.sync_copy(src_hbm.at[pl.ds(ebase + s * SBE, SBE)], gidx)
        pltpu.sync_copy(attr_hbm.at[pl.ds(ebase + s * SBE, SBE)], attrb)
        pltpu.sync_copy(dst4_hbm.at[sid, s], dstb)

        @pl.loop(0, SBE // LANES)
        def _mkidx(j):
            sl = pl.ds(j * LANES, LANES)
            gidx[sl] = gidx[sl] + c

        sems = (gsem0, gsem1, gsem2, gsem3)
        for l in range(NSLOT):
            gather_start(l, l, sems[l])

        @pl.loop(0, T)
        def _quad(t):
            u = NSLOT * t
            for l in range(NSLOT):
                gather_wait(u + l, l, sems[l])
                scale(u + l, l)
                scatter_sync(u + l, l)

                @pl.when(u + l + NSLOT < SB)
                def _pref():
                    gather_start(u + l + NSLOT, l, sems[l])

        gather_wait(SB - 1, 0, gsem0)
        scale(SB - 1, 0)
        scatter_sync(SB - 1, 0)

    plsc.subcore_barrier()

    w = c * NS + sid
    pltpu.sync_copy(agg_sh.at[pl.ds(sid * ROWS_PER_TILE, ROWS_PER_TILE)],
                    out_hbm.at[w])


def _sc_agg(x2, src, dst4, attr):
    mesh = plsc.VectorSubcoreMesh(core_axis_name="c", subcore_axis_name="s")
    kern = pl.kernel(
        _sc_agg_body,
        out_type=jax.ShapeDtypeStruct((NC * NS, ROWS_PER_TILE, DH),
                                      jnp.float32),
        mesh=mesh,
        scratch_types=[
            pltpu.VMEM((SBE,), jnp.int32),
            pltpu.VMEM((SB, 1, K), jnp.int32),
            pltpu.VMEM((SBE,), jnp.float32),
            pltpu.VMEM((NSLOT * K, DH), jnp.float32),
            pltpu.VMEM_SHARED((N_NODES, DH), jnp.float32),
            pltpu.SemaphoreType.DMA,
            pltpu.SemaphoreType.DMA,
            pltpu.SemaphoreType.DMA,
            pltpu.SemaphoreType.DMA,
        ],
    )
    return kern(x2, src, dst4, attr)


def _tc_body(x_ref, a0_ref, a1_ref, wroot_ref, wr0_ref, wr1_ref, b_ref,
             o_ref):
    acc = jnp.dot(x_ref[...], wroot_ref[...],
                  preferred_element_type=jnp.float32)
    acc += jnp.dot(a0_ref[...], wr0_ref[...],
                   preferred_element_type=jnp.float32)
    acc += jnp.dot(a1_ref[...], wr1_ref[...],
                   preferred_element_type=jnp.float32)
    o_ref[...] = jnp.maximum(acc + b_ref[...], 0.0)


def _tc_linear(x, a0, a1, wroot_t, wr0, wr1, b2):
    br = 1000
    grid = (N_NODES // br,)
    return pl.pallas_call(
        _tc_body,
        grid=grid,
        in_specs=[
            pl.BlockSpec((br, D_IN), lambda i: (i, 0)),
            pl.BlockSpec((br, DH), lambda i: (i, 0)),
            pl.BlockSpec((br, DH), lambda i: (i, 0)),
            pl.BlockSpec((D_IN, D_OUT), lambda i: (0, 0)),
            pl.BlockSpec((DH, D_OUT), lambda i: (0, 0)),
            pl.BlockSpec((DH, D_OUT), lambda i: (0, 0)),
            pl.BlockSpec((1, D_OUT), lambda i: (0, 0)),
        ],
        out_specs=pl.BlockSpec((br, D_OUT), lambda i: (i, 0)),
        out_shape=jax.ShapeDtypeStruct((N_NODES, D_OUT), jnp.float32),
    )(x, a0, a1, wroot_t, wr0, wr1, b2)


@jax.jit
def kernel(x, edge_index, edge_attr, W_rel, b_rel, W_root):
    src = edge_index[0].astype(jnp.int32) * 2
    dst = edge_index[1].astype(jnp.int32)
    x2 = x.reshape(2 * N_NODES, DH)
    dst4 = dst.reshape(NS, NSB, SB, 1, K)

    agg = _sc_agg(x2, src, dst4, edge_attr)
    a0 = agg[:NS].reshape(N_NODES, DH)
    a1 = agg[NS:].reshape(N_NODES, DH)

    wroot_t = W_root.T
    wr0 = W_rel[:, :DH].T
    wr1 = W_rel[:, DH:].T
    b2 = b_rel[None, :]
    return _tc_linear(x, a0, a1, wroot_t, wr0, wr1, b2)
